# baseline (speedup 1.0000x reference)
import functools

import jax
import jax.numpy as jnp
from jax.experimental import pallas as pl
from jax.experimental.pallas import tpu as pltpu

B, H, S, D = 8, 16, 2048, 128
Q = 16
BH = B * H


def _copy_scatter_kernel(pos_ref, kc_ref, vc_ref, kv_ref, vv_ref,
                         ok_ref, ov_ref, sem):
    p0 = pos_ref[0]
    contig = functools.reduce(
        jnp.logical_and,
        [pos_ref[i] == p0 + i for i in range(1, Q)])

    @pl.when(contig)
    def _():
        kc_ref[0, pl.ds(p0, Q), :] = kv_ref[0]
        vc_ref[0, pl.ds(p0, Q), :] = vv_ref[0]

    @pl.when(jnp.logical_not(contig))
    def _():
        for i in range(Q):
            p = pos_ref[i]
            kc_ref[0, pl.ds(p, 1), :] = kv_ref[0, pl.ds(i, 1), :]
            vc_ref[0, pl.ds(p, 1), :] = vv_ref[0, pl.ds(i, 1), :]

    i = pl.program_id(0)
    dk = pltpu.make_async_copy(kc_ref, ok_ref.at[pl.ds(i, 1)], sem.at[0])
    dv = pltpu.make_async_copy(vc_ref, ov_ref.at[pl.ds(i, 1)], sem.at[1])
    dk.start()
    dv.start()
    dk.wait()
    dv.wait()


def kernel(k_cache, v_cache, input_pos, k_val, v_val):
    kc = k_cache.reshape(BH, S, D)
    vc = v_cache.reshape(BH, S, D)
    kv = k_val.reshape(BH, Q, D)
    vv = v_val.reshape(BH, Q, D)

    out_k, out_v = pl.pallas_call(
        _copy_scatter_kernel,
        grid=(BH,),
        out_shape=[jax.ShapeDtypeStruct((BH, S, D), jnp.float32)] * 2,
        in_specs=[
            pl.BlockSpec(memory_space=pltpu.SMEM),
            pl.BlockSpec((1, S, D), lambda i: (i, 0, 0)),
            pl.BlockSpec((1, S, D), lambda i: (i, 0, 0)),
            pl.BlockSpec((1, Q, D), lambda i: (i, 0, 0)),
            pl.BlockSpec((1, Q, D), lambda i: (i, 0, 0)),
        ],
        out_specs=[pl.BlockSpec(memory_space=pl.ANY)] * 2,
        scratch_shapes=[pltpu.SemaphoreType.DMA((2,))],
        compiler_params=pltpu.CompilerParams(
            dimension_semantics=("arbitrary",)),
    )(input_pos, kc, vc, kv, vv)
    return (out_k.reshape(B, H, S, D), out_v.reshape(B, H, S, D))


# speedup vs baseline: 1.4584x; 1.4584x over previous
import functools

import jax
import jax.numpy as jnp
from jax.experimental import pallas as pl
from jax.experimental.pallas import tpu as pltpu

B, H, S, D = 8, 16, 2048, 128
Q = 16
BH = B * H
BLK = 2


def _copy_scatter_kernel(pos_ref, kc_ref, vc_ref, kv_ref, vv_ref,
                         ok_ref, ov_ref):
    ok_ref[...] = kc_ref[...]
    ov_ref[...] = vc_ref[...]

    p0 = pos_ref[0]
    contig = functools.reduce(
        jnp.logical_and,
        [pos_ref[i] == p0 + i for i in range(1, Q)])

    @pl.when(contig)
    def _():
        for b in range(BLK):
            ok_ref[b, pl.ds(p0, Q), :] = kv_ref[b]
            ov_ref[b, pl.ds(p0, Q), :] = vv_ref[b]

    @pl.when(jnp.logical_not(contig))
    def _():
        for b in range(BLK):
            for i in range(Q):
                p = pos_ref[i]
                ok_ref[b, pl.ds(p, 1), :] = kv_ref[b, pl.ds(i, 1), :]
                ov_ref[b, pl.ds(p, 1), :] = vv_ref[b, pl.ds(i, 1), :]


def kernel(k_cache, v_cache, input_pos, k_val, v_val):
    kc = k_cache.reshape(BH, S, D)
    vc = v_cache.reshape(BH, S, D)
    kv = k_val.reshape(BH, Q, D)
    vv = v_val.reshape(BH, Q, D)

    out_k, out_v = pl.pallas_call(
        _copy_scatter_kernel,
        grid=(BH // BLK,),
        out_shape=[jax.ShapeDtypeStruct((BH, S, D), jnp.float32)] * 2,
        in_specs=[
            pl.BlockSpec(memory_space=pltpu.SMEM),
            pl.BlockSpec((BLK, S, D), lambda i: (i, 0, 0)),
            pl.BlockSpec((BLK, S, D), lambda i: (i, 0, 0)),
            pl.BlockSpec((BLK, Q, D), lambda i: (i, 0, 0)),
            pl.BlockSpec((BLK, Q, D), lambda i: (i, 0, 0)),
        ],
        out_specs=[pl.BlockSpec((BLK, S, D), lambda i: (i, 0, 0))] * 2,
        compiler_params=pltpu.CompilerParams(
            dimension_semantics=("arbitrary",)),
    )(input_pos, kc, vc, kv, vv)
    return (out_k.reshape(B, H, S, D), out_v.reshape(B, H, S, D))


# speedup vs baseline: 1.4840x; 1.0175x over previous
import functools

import jax
import jax.numpy as jnp
from jax.experimental import pallas as pl
from jax.experimental.pallas import tpu as pltpu

B, H, S, D = 8, 16, 2048, 128
Q = 16
BH = B * H
BLK = 4


def _copy_scatter_kernel(pos_ref, kc_ref, vc_ref, kv_ref, vv_ref,
                         ok_ref, ov_ref):
    ok_ref[...] = kc_ref[...]
    ov_ref[...] = vc_ref[...]

    p0 = pos_ref[0]
    contig = functools.reduce(
        jnp.logical_and,
        [pos_ref[i] == p0 + i for i in range(1, Q)])

    @pl.when(contig)
    def _():
        for b in range(BLK):
            ok_ref[b, pl.ds(p0, Q), :] = kv_ref[b]
            ov_ref[b, pl.ds(p0, Q), :] = vv_ref[b]

    @pl.when(jnp.logical_not(contig))
    def _():
        for b in range(BLK):
            for i in range(Q):
                p = pos_ref[i]
                ok_ref[b, pl.ds(p, 1), :] = kv_ref[b, pl.ds(i, 1), :]
                ov_ref[b, pl.ds(p, 1), :] = vv_ref[b, pl.ds(i, 1), :]


def kernel(k_cache, v_cache, input_pos, k_val, v_val):
    kc = k_cache.reshape(BH, S, D)
    vc = v_cache.reshape(BH, S, D)
    kv = k_val.reshape(BH, Q, D)
    vv = v_val.reshape(BH, Q, D)

    out_k, out_v = pl.pallas_call(
        _copy_scatter_kernel,
        grid=(BH // BLK,),
        out_shape=[jax.ShapeDtypeStruct((BH, S, D), jnp.float32)] * 2,
        in_specs=[
            pl.BlockSpec(memory_space=pltpu.SMEM),
            pl.BlockSpec((BLK, S, D), lambda i: (i, 0, 0)),
            pl.BlockSpec((BLK, S, D), lambda i: (i, 0, 0)),
            pl.BlockSpec((BLK, Q, D), lambda i: (i, 0, 0)),
            pl.BlockSpec((BLK, Q, D), lambda i: (i, 0, 0)),
        ],
        out_specs=[pl.BlockSpec((BLK, S, D), lambda i: (i, 0, 0))] * 2,
        compiler_params=pltpu.CompilerParams(
            dimension_semantics=("arbitrary",)),
    )(input_pos, kc, vc, kv, vv)
    return (out_k.reshape(B, H, S, D), out_v.reshape(B, H, S, D))
